# Initial kernel scaffold; baseline (speedup 1.0000x reference)
import jax
import jax.numpy as jnp
from jax.experimental import pallas as pl


def kernel(x):
    raise NotImplementedError("write your pallas kernel here")


# speedup vs baseline: 13.1562x; 13.1562x over previous
import functools

import jax
import jax.numpy as jnp
from jax import lax
from jax.experimental import pallas as pl
from jax.experimental.pallas import tpu as pltpu
from jax.experimental.pallas import tpu_sc as plsc

L = 16
NROWS = 128
DIM = 32768
NCHUNK = DIM // L
NC = 2
NS = 16
NW = NC * NS
ROWS_PER = NROWS // NW
CAP = 16376
FAST_MAX = CAP - L
BISECT_ITERS = 24


_mesh = plsc.VectorSubcoreMesh(
    core_axis_name="c", subcore_axis_name="s", num_cores=NC, num_subcores=NS
)


@functools.partial(
    pl.kernel,
    out_type=jax.ShapeDtypeStruct((NROWS, DIM), jnp.float32),
    mesh=_mesh,
    scratch_types=[
        pltpu.VMEM((DIM,), jnp.float32),
        pltpu.VMEM((DIM,), jnp.float32),
        pltpu.VMEM((DIM,), jnp.float32),
        pltpu.VMEM((CAP,), jnp.float32),
        pltpu.VMEM((CAP,), jnp.int32),
        pltpu.SemaphoreType.DMA,
        pltpu.SemaphoreType.DMA,
        pltpu.SemaphoreType.DMA,
    ],
    compiler_params=pltpu.CompilerParams(needs_layout_passes=False),
)
def _sparsemax_sc(x_hbm, o_hbm, in0, in1, ob, cand_v, cand_i, s0, s1, so):
    iota = lax.iota(jnp.int32, L)
    vzero = jnp.zeros((L,), jnp.float32)

    def row_max(buf):
        def step(i, acc):
            return jnp.maximum(acc, buf[pl.ds(i * L, L)])

        acc = lax.fori_loop(
            0, NCHUNK, step, jnp.full((L,), -jnp.inf, jnp.float32), unroll=8
        )
        return jnp.max(acc)

    def compress(buf, cutoff):
        def step(i, off):
            v = buf[pl.ds(i * L, L)]
            mask = v > cutoff
            ok = off <= (CAP - L)
            base = jnp.minimum(off, CAP - L)
            smask = jnp.logical_and(mask, ok)
            plsc.store_compressed(cand_v.at[pl.ds(base, L)], v, mask=smask)
            plsc.store_compressed(
                cand_i.at[pl.ds(base, L)], iota + i * L, mask=smask
            )
            return off + jnp.sum(mask.astype(jnp.int32))

        return lax.fori_loop(0, NCHUNK, step, jnp.int32(0), unroll=2)

    def nchunks(m):
        return (m + (L - 1)) >> 4

    def fdiv(a, b):
        av = jnp.broadcast_to(a, (L,))
        bv = jnp.broadcast_to(b, (L,))
        return (av / bv)[0]

    def g_cand(m, t):
        def step(i, acc):
            v = cand_v[pl.ds(i * L, L)]
            valid = (iota + i * L) < m
            return acc + jnp.where(valid, jnp.maximum(v - t, 0.0), 0.0)

        acc = lax.fori_loop(0, nchunks(m), step, vzero)
        return jnp.sum(acc)

    def ks_cand(m, lo):
        def step(i, acc):
            ka, sa = acc
            v = cand_v[pl.ds(i * L, L)]
            sel = jnp.logical_and((iota + i * L) < m, v > lo)
            return (
                ka + jnp.where(sel, 1.0, 0.0),
                sa + jnp.where(sel, v, 0.0),
            )

        ka, sa = lax.fori_loop(0, nchunks(m), step, (vzero, vzero))
        return jnp.sum(ka), jnp.sum(sa)

    def g_dense(buf, t):
        def step(i, acc):
            v = buf[pl.ds(i * L, L)]
            return acc + jnp.maximum(v - t, 0.0)

        acc = lax.fori_loop(0, NCHUNK, step, vzero, unroll=4)
        return jnp.sum(acc)

    def ks_dense(buf, lo):
        def step(i, acc):
            ka, sa = acc
            v = buf[pl.ds(i * L, L)]
            sel = v > lo
            return (
                ka + jnp.where(sel, 1.0, 0.0),
                sa + jnp.where(sel, v, 0.0),
            )

        ka, sa = lax.fori_loop(0, NCHUNK, step, (vzero, vzero))
        return jnp.sum(ka), jnp.sum(sa)

    def solve(g_fn, ks_fn, lo0, hi0):
        def bis(_, lh):
            lo, hi = lh
            mid = 0.5 * (lo + hi)
            ge = g_fn(mid) >= 1.0
            return (jnp.where(ge, mid, lo), jnp.where(ge, hi, mid))

        lo, _ = lax.fori_loop(0, BISECT_ITERS, bis, (lo0, hi0))
        kk, ss = ks_fn(lo)
        return fdiv(ss - 1.0, kk)

    def scatter_vals(m, t):
        def step(i, _):
            v = cand_v[pl.ds(i * L, L)]
            ix = cand_i[pl.ds(i * L, L)]
            valid = (iota + i * L) < m
            ix = jnp.where(valid, ix, 0)
            plsc.store_scatter(
                ob, [ix], jnp.maximum(v - t, 0.0), mask=valid
            )
            return jnp.int32(0)

        lax.fori_loop(0, nchunks(m), step, jnp.int32(0))

    def scatter_zeros(m):
        def step(i, _):
            ix = cand_i[pl.ds(i * L, L)]
            valid = (iota + i * L) < m
            ix = jnp.where(valid, ix, 0)
            plsc.store_scatter(ob, [ix], vzero, mask=valid)
            return jnp.int32(0)

        lax.fori_loop(0, nchunks(m), step, jnp.int32(0))

    def dense_zero():
        def step(i, _):
            ob[pl.ds(i * L, L)] = vzero
            return jnp.int32(0)

        lax.fori_loop(0, NCHUNK, step, jnp.int32(0), unroll=8)

    def dense_relu(buf, t):
        def step(i, _):
            ob[pl.ds(i * L, L)] = jnp.maximum(buf[pl.ds(i * L, L)] - t, 0.0)
            return jnp.int32(0)

        lax.fori_loop(0, NCHUNK, step, jnp.int32(0), unroll=4)

    wid = lax.axis_index("s") * NC + lax.axis_index("c")
    row0 = wid * ROWS_PER
    bufs = (in0, in1)
    sems = (s0, s1)

    in_handles = {0: pltpu.async_copy(x_hbm.at[row0], in0, s0)}
    out_handle = None
    prev_dense = None
    prev_m = jnp.int32(0)

    for j in range(ROWS_PER):
        buf = bufs[j % 2]
        in_handles[j].wait()
        if j + 1 < ROWS_PER:
            in_handles[j + 1] = pltpu.async_copy(
                x_hbm.at[row0 + j + 1], bufs[(j + 1) % 2], sems[(j + 1) % 2]
            )

        xmax = row_max(buf)
        cutoff = xmax - 1.0

        if out_handle is not None:
            out_handle.wait()
        if prev_dense is None:
            dense_zero()
        else:
            lax.cond(
                prev_dense,
                lambda pm: dense_zero(),
                lambda pm: scatter_zeros(pm),
                prev_m,
            )

        m = compress(buf, cutoff)
        fast = m <= FAST_MAX

        def fast_fn(opnd, buf=buf):
            mm, cut, xm = opnd
            t = solve(
                lambda tt: g_cand(mm, tt), lambda lo: ks_cand(mm, lo), cut, xm
            )
            scatter_vals(mm, t)
            return jnp.float32(0)

        def slow_fn(opnd, buf=buf):
            mm, cut, xm = opnd
            t = solve(
                lambda tt: g_dense(buf, tt),
                lambda lo: ks_dense(buf, lo),
                cut,
                xm,
            )
            dense_relu(buf, t)
            return jnp.float32(0)

        lax.cond(fast, fast_fn, slow_fn, (m, cutoff, xmax))

        out_handle = pltpu.async_copy(ob, o_hbm.at[row0 + j], so)
        prev_dense = jnp.logical_not(fast)
        prev_m = m

    out_handle.wait()


def kernel(x):
    return _sparsemax_sc(x)


# speedup vs baseline: 14.5671x; 1.1072x over previous
import functools

import jax
import jax.numpy as jnp
from jax import lax
from jax.experimental import pallas as pl
from jax.experimental.pallas import tpu as pltpu
from jax.experimental.pallas import tpu_sc as plsc

L = 16
NROWS = 128
DIM = 32768
NCHUNK = DIM // L
G = 16
NG = NCHUNK // G
NC = 2
NS = 16
NW = NC * NS
ROWS_PER = NROWS // NW
CAP = 16376
FAST_MAX = CAP - L
BISECT_ITERS = 24


_mesh = plsc.VectorSubcoreMesh(
    core_axis_name="c", subcore_axis_name="s", num_cores=NC, num_subcores=NS
)


@functools.partial(
    pl.kernel,
    out_type=jax.ShapeDtypeStruct((NROWS, DIM), jnp.float32),
    mesh=_mesh,
    scratch_types=[
        pltpu.VMEM((DIM,), jnp.float32),
        pltpu.VMEM((DIM,), jnp.float32),
        pltpu.VMEM((DIM,), jnp.float32),
        pltpu.VMEM((CAP,), jnp.float32),
        pltpu.VMEM((NG * L,), jnp.float32),
        pltpu.SemaphoreType.DMA,
        pltpu.SemaphoreType.DMA,
        pltpu.SemaphoreType.DMA,
    ],
    compiler_params=pltpu.CompilerParams(needs_layout_passes=False),
)
def _sparsemax_sc(x_hbm, o_hbm, in0, in1, ob, cand_v, gmax, s0, s1, so):
    iota = lax.iota(jnp.int32, L)
    vzero = jnp.zeros((L,), jnp.float32)

    def phase_a(buf):
        def gstep(g, acc):
            vs = [buf[pl.ds((g * G + c) * L, L)] for c in range(G)]
            while len(vs) > 1:
                vs = [
                    jnp.maximum(vs[i], vs[i + 1]) for i in range(0, len(vs), 2)
                ]
            gmax[pl.ds(g * L, L)] = vs[0]
            return jnp.maximum(acc, vs[0])

        acc = lax.fori_loop(
            0, NG, gstep, jnp.full((L,), -jnp.inf, jnp.float32)
        )
        return jnp.max(acc)

    def compress(buf, cutoff):
        def gstep(g, off):
            anyv = jnp.any(gmax[pl.ds(g * L, L)] > cutoff)

            def fire(o):
                def cstep(c, o2):
                    v = buf[pl.ds((g * G + c) * L, L)]
                    mask = v > cutoff
                    cnt = plsc.all_reduce_population_count(mask)[0]
                    ok = o2 <= (CAP - L)
                    base = jnp.minimum(o2, CAP - L)
                    plsc.store_compressed(
                        cand_v.at[pl.ds(base, L)],
                        v,
                        mask=jnp.logical_and(mask, ok),
                    )
                    return o2 + cnt

                return lax.fori_loop(0, G, cstep, o, unroll=4)

            return lax.cond(anyv, fire, lambda o: o, off)

        return lax.fori_loop(0, NG, gstep, jnp.int32(0))

    def nchunks(m):
        return (m + (L - 1)) >> 4

    def fdiv(a, b):
        av = jnp.broadcast_to(a, (L,))
        bv = jnp.broadcast_to(b, (L,))
        return (av / bv)[0]

    def g_cand(m, t):
        def step(i, acc):
            v = cand_v[pl.ds(i * L, L)]
            valid = (iota + i * L) < m
            return acc + jnp.where(valid, jnp.maximum(v - t, 0.0), 0.0)

        acc = lax.fori_loop(0, nchunks(m), step, vzero)
        return jnp.sum(acc)

    def ks_cand(m, lo):
        def step(i, acc):
            ka, sa = acc
            v = cand_v[pl.ds(i * L, L)]
            sel = jnp.logical_and((iota + i * L) < m, v > lo)
            return (
                ka + jnp.where(sel, 1.0, 0.0),
                sa + jnp.where(sel, v, 0.0),
            )

        ka, sa = lax.fori_loop(0, nchunks(m), step, (vzero, vzero))
        return jnp.sum(ka), jnp.sum(sa)

    def g_dense(buf, t):
        def step(i, acc):
            v = buf[pl.ds(i * L, L)]
            return acc + jnp.maximum(v - t, 0.0)

        acc = lax.fori_loop(0, NCHUNK, step, vzero, unroll=4)
        return jnp.sum(acc)

    def ks_dense(buf, lo):
        def step(i, acc):
            ka, sa = acc
            v = buf[pl.ds(i * L, L)]
            sel = v > lo
            return (
                ka + jnp.where(sel, 1.0, 0.0),
                sa + jnp.where(sel, v, 0.0),
            )

        ka, sa = lax.fori_loop(0, NCHUNK, step, (vzero, vzero))
        return jnp.sum(ka), jnp.sum(sa)

    def solve(g_fn, ks_fn, lo0, hi0):
        def bis(_, lh):
            lo, hi = lh
            mid = 0.5 * (lo + hi)
            ge = g_fn(mid) >= 1.0
            return (jnp.where(ge, mid, lo), jnp.where(ge, hi, mid))

        lo, _ = lax.fori_loop(0, BISECT_ITERS, bis, (lo0, hi0))
        kk, ss = ks_fn(lo)
        return fdiv(ss - 1.0, kk)

    def dense_relu(buf, t):
        def step(i, _):
            ob[pl.ds(i * L, L)] = jnp.maximum(buf[pl.ds(i * L, L)] - t, 0.0)
            return jnp.int32(0)

        lax.fori_loop(0, NCHUNK, step, jnp.int32(0), unroll=8)

    wid = lax.axis_index("s") * NC + lax.axis_index("c")
    row0 = wid * ROWS_PER
    bufs = (in0, in1)
    sems = (s0, s1)

    in_handles = {0: pltpu.async_copy(x_hbm.at[row0], in0, s0)}
    out_handle = None

    for j in range(ROWS_PER):
        buf = bufs[j % 2]
        in_handles[j].wait()
        if j + 1 < ROWS_PER:
            in_handles[j + 1] = pltpu.async_copy(
                x_hbm.at[row0 + j + 1], bufs[(j + 1) % 2], sems[(j + 1) % 2]
            )

        xmax = phase_a(buf)
        cutoff = xmax - 1.0
        m = compress(buf, cutoff)
        fast = m <= FAST_MAX

        def fast_fn(opnd):
            mm, cut, xm = opnd
            return solve(
                lambda tt: g_cand(mm, tt), lambda lo: ks_cand(mm, lo), cut, xm
            )

        def slow_fn(opnd, buf=buf):
            mm, cut, xm = opnd
            return solve(
                lambda tt: g_dense(buf, tt),
                lambda lo: ks_dense(buf, lo),
                cut,
                xm,
            )

        t = lax.cond(fast, fast_fn, slow_fn, (m, cutoff, xmax))

        if out_handle is not None:
            out_handle.wait()
        dense_relu(buf, t)
        out_handle = pltpu.async_copy(ob, o_hbm.at[row0 + j], so)

    out_handle.wait()


def kernel(x):
    return _sparsemax_sc(x)


# speedup vs baseline: 29.3970x; 2.0180x over previous
import functools

import jax
import jax.numpy as jnp
from jax import lax
from jax.experimental import pallas as pl
from jax.experimental.pallas import tpu as pltpu
from jax.experimental.pallas import tpu_sc as plsc

L = 16
NROWS = 128
DIM = 32768
NCHUNK = DIM // L
G = 16
NG = NCHUNK // G
NC = 2
NS = 16
NW = NC * NS
ROWS_PER = NROWS // NW
CAP = 16376
FAST_MAX = CAP - L
BISECT_ITERS = 24


_mesh = plsc.VectorSubcoreMesh(
    core_axis_name="c", subcore_axis_name="s", num_cores=NC, num_subcores=NS
)


@functools.partial(
    pl.kernel,
    out_type=jax.ShapeDtypeStruct((NROWS, DIM), jnp.float32),
    mesh=_mesh,
    scratch_types=[
        pltpu.VMEM((DIM,), jnp.float32),
        pltpu.VMEM((DIM,), jnp.float32),
        pltpu.VMEM((DIM,), jnp.float32),
        pltpu.VMEM((CAP,), jnp.float32),
        pltpu.VMEM((NG * L,), jnp.float32),
        pltpu.SemaphoreType.DMA,
        pltpu.SemaphoreType.DMA,
        pltpu.SemaphoreType.DMA,
    ],
    compiler_params=pltpu.CompilerParams(needs_layout_passes=False),
)
def _sparsemax_sc(x_hbm, o_hbm, in0, in1, ob, cand_v, gmax, s0, s1, so):
    iota = lax.iota(jnp.int32, L)
    vzero = jnp.zeros((L,), jnp.float32)

    def phase_a(buf):
        def gstep(g, acc):
            vs = [buf[pl.ds((g * G + c) * L, L)] for c in range(G)]
            while len(vs) > 1:
                vs = [
                    jnp.maximum(vs[i], vs[i + 1]) for i in range(0, len(vs), 2)
                ]
            gmax[pl.ds(g * L, L)] = vs[0]
            return jnp.maximum(acc, vs[0])

        acc = plsc.parallel_loop(
            0, NG, 1, unroll=2, carry=jnp.full((L,), -jnp.inf, jnp.float32)
        )(gstep)
        return jnp.max(acc)

    def compress(buf, cutoff):
        def gstep(g, off):
            anyv = jnp.any(gmax[pl.ds(g * L, L)] > cutoff)

            def fire(o):
                def cstep(c, o2):
                    v = buf[pl.ds((g * G + c) * L, L)]
                    mask = v > cutoff
                    cnt = plsc.all_reduce_population_count(mask)[0]
                    ok = o2 <= (CAP - L)
                    base = jnp.minimum(o2, CAP - L)
                    plsc.store_compressed(
                        cand_v.at[pl.ds(base, L)],
                        v,
                        mask=jnp.logical_and(mask, ok),
                    )
                    return o2 + cnt

                return plsc.parallel_loop(0, G, 1, unroll=4, carry=o)(cstep)

            return lax.cond(anyv, fire, lambda o: o, off)

        return plsc.parallel_loop(0, NG, 1, carry=jnp.int32(0))(gstep)

    def nchunks(m):
        return (m + (L - 1)) >> 4

    def fdiv(a, b):
        av = jnp.broadcast_to(a, (L,))
        bv = jnp.broadcast_to(b, (L,))
        return (av / bv)[0]

    def g_cand(m, t):
        def step(i, acc):
            v = cand_v[pl.ds(i * L, L)]
            valid = (iota + i * L) < m
            return acc + jnp.where(valid, jnp.maximum(v - t, 0.0), 0.0)

        acc = lax.fori_loop(0, nchunks(m), step, vzero)
        return jnp.sum(acc)

    def ks_cand(m, lo):
        def step(i, acc):
            ka, sa = acc
            v = cand_v[pl.ds(i * L, L)]
            sel = jnp.logical_and((iota + i * L) < m, v > lo)
            return (
                ka + jnp.where(sel, 1.0, 0.0),
                sa + jnp.where(sel, v, 0.0),
            )

        ka, sa = lax.fori_loop(0, nchunks(m), step, (vzero, vzero))
        return jnp.sum(ka), jnp.sum(sa)

    def g_dense(buf, t):
        def step(i, acc):
            v = buf[pl.ds(i * L, L)]
            return acc + jnp.maximum(v - t, 0.0)

        acc = lax.fori_loop(0, NCHUNK, step, vzero, unroll=4)
        return jnp.sum(acc)

    def ks_dense(buf, lo):
        def step(i, acc):
            ka, sa = acc
            v = buf[pl.ds(i * L, L)]
            sel = v > lo
            return (
                ka + jnp.where(sel, 1.0, 0.0),
                sa + jnp.where(sel, v, 0.0),
            )

        ka, sa = lax.fori_loop(0, NCHUNK, step, (vzero, vzero))
        return jnp.sum(ka), jnp.sum(sa)

    def solve(g_fn, ks_fn, lo0, hi0):
        def bis(_, lh):
            lo, hi = lh
            mid = 0.5 * (lo + hi)
            ge = g_fn(mid) >= 1.0
            return (jnp.where(ge, mid, lo), jnp.where(ge, hi, mid))

        lo, _ = lax.fori_loop(0, BISECT_ITERS, bis, (lo0, hi0))
        kk, ss = ks_fn(lo)
        return fdiv(ss - 1.0, kk)

    def dense_relu(buf, t):
        def step(i):
            ob[pl.ds(i * L, L)] = jnp.maximum(buf[pl.ds(i * L, L)] - t, 0.0)

        plsc.parallel_loop(0, NCHUNK, 1, unroll=8)(step)

    wid = lax.axis_index("s") * NC + lax.axis_index("c")
    row0 = wid * ROWS_PER
    bufs = (in0, in1)
    sems = (s0, s1)

    in_handles = {0: pltpu.async_copy(x_hbm.at[row0], in0, s0)}
    out_handle = None

    for j in range(ROWS_PER):
        buf = bufs[j % 2]
        in_handles[j].wait()
        if j + 1 < ROWS_PER:
            in_handles[j + 1] = pltpu.async_copy(
                x_hbm.at[row0 + j + 1], bufs[(j + 1) % 2], sems[(j + 1) % 2]
            )

        xmax = phase_a(buf)
        cutoff = xmax - 1.0
        m = compress(buf, cutoff)
        fast = m <= FAST_MAX

        def fast_fn(opnd):
            mm, cut, xm = opnd
            return solve(
                lambda tt: g_cand(mm, tt), lambda lo: ks_cand(mm, lo), cut, xm
            )

        def slow_fn(opnd, buf=buf):
            mm, cut, xm = opnd
            return solve(
                lambda tt: g_dense(buf, tt),
                lambda lo: ks_dense(buf, lo),
                cut,
                xm,
            )

        t = lax.cond(fast, fast_fn, slow_fn, (m, cutoff, xmax))

        if out_handle is not None:
            out_handle.wait()
        dense_relu(buf, t)
        out_handle = pltpu.async_copy(ob, o_hbm.at[row0 + j], so)

    out_handle.wait()


def kernel(x):
    return _sparsemax_sc(x)


# speedup vs baseline: 34.3094x; 1.1671x over previous
import functools

import jax
import jax.numpy as jnp
from jax import lax
from jax.experimental import pallas as pl
from jax.experimental.pallas import tpu as pltpu
from jax.experimental.pallas import tpu_sc as plsc

L = 16
NROWS = 128
DIM = 32768
NCHUNK = DIM // L
NC = 2
NS = 16
NW = NC * NS
ROWS_PER = NROWS // NW
CAP = 16376
FAST_MAX = CAP - L
BISECT_ITERS = 24


_mesh = plsc.VectorSubcoreMesh(
    core_axis_name="c", subcore_axis_name="s", num_cores=NC, num_subcores=NS
)


@functools.partial(
    pl.kernel,
    out_type=jax.ShapeDtypeStruct((NROWS, DIM), jnp.float32),
    mesh=_mesh,
    scratch_types=[
        pltpu.VMEM((DIM,), jnp.float32),
        pltpu.VMEM((DIM,), jnp.float32),
        pltpu.VMEM((DIM,), jnp.float32),
        pltpu.VMEM((CAP,), jnp.float32),
        pltpu.SemaphoreType.DMA,
        pltpu.SemaphoreType.DMA,
        pltpu.SemaphoreType.DMA,
    ],
    compiler_params=pltpu.CompilerParams(needs_layout_passes=False),
)
def _sparsemax_sc(x_hbm, o_hbm, in0, in1, ob, cand_v, s0, s1, so):
    vzero = jnp.zeros((L,), jnp.float32)
    vone_i = jnp.ones((L,), jnp.int32)

    def phase_a(buf):
        def gstep(g, acc):
            vs = [buf[pl.ds((g * 16 + c) * L, L)] for c in range(16)]
            while len(vs) > 1:
                vs = [
                    jnp.maximum(vs[i], vs[i + 1]) for i in range(0, len(vs), 2)
                ]
            return jnp.maximum(acc, vs[0])

        acc = plsc.parallel_loop(
            0, NCHUNK // 16, 1, unroll=2,
            carry=jnp.full((L,), -jnp.inf, jnp.float32),
        )(gstep)
        return jnp.max(acc)

    def compress(buf, cutoff):
        def step(i, off_v):
            v = buf[pl.ds(i * L, L)]
            mask = v > cutoff
            ones = jnp.where(mask, vone_i, 0)
            prefix = plsc.cumsum(ones)
            dest = jnp.minimum(off_v + prefix - 1, CAP - 1)
            plsc.store_scatter(cand_v, [dest], v, mask=mask)
            cnt = plsc.all_reduce_population_count(mask)
            return off_v + cnt

        off_v = plsc.parallel_loop(
            0, NCHUNK, 1, unroll=4, carry=jnp.zeros((L,), jnp.int32)
        )(step)
        return off_v[0]

    def nchunks(m):
        return (m + (L - 1)) >> 4

    def g_cand(iota, m, t):
        def step(i, acc):
            v = cand_v[pl.ds(i * L, L)]
            valid = (iota + i * L) < m
            return acc + jnp.where(valid, jnp.maximum(v - t, 0.0), 0.0)

        acc = lax.fori_loop(0, nchunks(m), step, vzero)
        return jnp.sum(acc)

    def ks_cand(iota, m, lo):
        def step(i, acc):
            ka, sa = acc
            v = cand_v[pl.ds(i * L, L)]
            sel = jnp.logical_and((iota + i * L) < m, v > lo)
            return (
                ka + jnp.where(sel, 1.0, 0.0),
                sa + jnp.where(sel, v, 0.0),
            )

        ka, sa = lax.fori_loop(0, nchunks(m), step, (vzero, vzero))
        return jnp.sum(ka), jnp.sum(sa)

    def g_dense(buf, t):
        def step(i, acc):
            v = buf[pl.ds(i * L, L)]
            return acc + jnp.maximum(v - t, 0.0)

        acc = lax.fori_loop(0, NCHUNK, step, vzero, unroll=4)
        return jnp.sum(acc)

    def ks_dense(buf, lo):
        def step(i, acc):
            ka, sa = acc
            v = buf[pl.ds(i * L, L)]
            sel = v > lo
            return (
                ka + jnp.where(sel, 1.0, 0.0),
                sa + jnp.where(sel, v, 0.0),
            )

        ka, sa = lax.fori_loop(0, NCHUNK, step, (vzero, vzero))
        return jnp.sum(ka), jnp.sum(sa)

    def solve(g_fn, ks_fn, lo0, hi0):
        def bis(_, lh):
            lo, hi = lh
            mid = 0.5 * (lo + hi)
            ge = g_fn(mid) >= 1.0
            return (jnp.where(ge, mid, lo), jnp.where(ge, hi, mid))

        lo, _ = lax.fori_loop(
            0,
            BISECT_ITERS,
            bis,
            (jnp.broadcast_to(lo0, (L,)), jnp.broadcast_to(hi0, (L,))),
        )
        kk, ss = ks_fn(lo)
        return (jnp.broadcast_to(ss, (L,)) - 1.0) / jnp.broadcast_to(kk, (L,))

    def dense_relu(buf, t):
        def step(i):
            ob[pl.ds(i * L, L)] = jnp.maximum(buf[pl.ds(i * L, L)] - t, 0.0)

        plsc.parallel_loop(0, NCHUNK, 1, unroll=8)(step)

    wid = lax.axis_index("s") * NC + lax.axis_index("c")
    row0 = wid * ROWS_PER
    bufs = (in0, in1)
    sems = (s0, s1)

    in_handles = {0: pltpu.async_copy(x_hbm.at[row0], in0, s0)}
    out_handle = None

    for j in range(ROWS_PER):
        buf = bufs[j % 2]
        in_handles[j].wait()
        if j + 1 < ROWS_PER:
            in_handles[j + 1] = pltpu.async_copy(
                x_hbm.at[row0 + j + 1], bufs[(j + 1) % 2], sems[(j + 1) % 2]
            )

        iota = lax.iota(jnp.int32, L)
        xmax = phase_a(buf)
        cutoff = xmax - 1.0
        cutoff_v = jnp.broadcast_to(cutoff, (L,))
        m = compress(buf, cutoff_v)
        fast = m <= FAST_MAX

        def fast_fn(opnd, iota=iota):
            mm, cut, xm = opnd
            return solve(
                lambda tt: g_cand(iota, mm, tt),
                lambda lo: ks_cand(iota, mm, lo),
                cut,
                xm,
            )

        def slow_fn(opnd, buf=buf):
            mm, cut, xm = opnd
            return solve(
                lambda tt: g_dense(buf, tt),
                lambda lo: ks_dense(buf, lo),
                cut,
                xm,
            )

        t = lax.cond(fast, fast_fn, slow_fn, (m, cutoff, xmax))

        if out_handle is not None:
            out_handle.wait()
        dense_relu(buf, t)
        out_handle = pltpu.async_copy(ob, o_hbm.at[row0 + j], so)

    out_handle.wait()


def kernel(x):
    return _sparsemax_sc(x)


# speedup vs baseline: 36.3163x; 1.0585x over previous
import functools

import jax
import jax.numpy as jnp
from jax import lax
from jax.experimental import pallas as pl
from jax.experimental.pallas import tpu as pltpu
from jax.experimental.pallas import tpu_sc as plsc

L = 16
NROWS = 128
DIM = 32768
NCHUNK = DIM // L
NC = 2
NS = 16
NW = NC * NS
ROWS_PER = NROWS // NW
CAP = 16376
FAST_MAX = CAP - L
BISECT_ITERS = 24


_mesh = plsc.VectorSubcoreMesh(
    core_axis_name="c", subcore_axis_name="s", num_cores=NC, num_subcores=NS
)


@functools.partial(
    pl.kernel,
    out_type=jax.ShapeDtypeStruct((NROWS, DIM), jnp.float32),
    mesh=_mesh,
    scratch_types=[
        pltpu.VMEM((DIM,), jnp.float32),
        pltpu.VMEM((DIM,), jnp.float32),
        pltpu.VMEM((DIM,), jnp.float32),
        pltpu.VMEM((CAP,), jnp.float32),
        pltpu.SemaphoreType.DMA,
        pltpu.SemaphoreType.DMA,
        pltpu.SemaphoreType.DMA,
    ],
    compiler_params=pltpu.CompilerParams(needs_layout_passes=False),
)
def _sparsemax_sc(x_hbm, o_hbm, in0, in1, ob, cand_v, s0, s1, so):
    vzero = jnp.zeros((L,), jnp.float32)
    vone_i = jnp.ones((L,), jnp.int32)

    def sampled_max(buf):
        def gstep(g, acc):
            a = jnp.maximum(buf[pl.ds(g * 8 * 2 * L, L)],
                            buf[pl.ds((g * 8 * 2 + 8) * L, L)])
            return jnp.maximum(acc, a)

        acc = plsc.parallel_loop(
            0, NCHUNK // 16, 1, unroll=4,
            carry=jnp.full((L,), -jnp.inf, jnp.float32),
        )(gstep)
        return jnp.max(acc)

    def compress(buf, cutoff):
        def step(i, carry):
            off_m1, mx = carry
            v = buf[pl.ds(i * L, L)]
            mask = v > cutoff
            prefix = plsc.cumsum(vone_i, mask=mask)
            dest = off_m1 + prefix
            dest = plsc.bitcast(
                jnp.minimum(
                    plsc.bitcast(dest, jnp.uint32), jnp.uint32(CAP - 1)
                ),
                jnp.int32,
            )
            plsc.store_scatter(cand_v, [dest], v, mask=mask)
            cnt = plsc.all_reduce_population_count(mask)
            return off_m1 + cnt, jnp.maximum(mx, v)

        off_m1, mx = plsc.parallel_loop(
            0, NCHUNK, 1, unroll=4,
            carry=(
                jnp.full((L,), -1, jnp.int32),
                jnp.full((L,), -jnp.inf, jnp.float32),
            ),
        )(step)
        return off_m1[0] + 1, jnp.max(mx)

    def nchunks(m):
        return (m + (L - 1)) >> 4

    def g_cand(iota, m, t):
        def step(i, acc):
            v = cand_v[pl.ds(i * L, L)]
            valid = (iota + i * L) < m
            return acc + jnp.where(valid, jnp.maximum(v - t, 0.0), 0.0)

        acc = lax.fori_loop(0, nchunks(m), step, vzero)
        return jnp.sum(acc)

    def ks_cand(iota, m, lo):
        def step(i, acc):
            ka, sa = acc
            v = cand_v[pl.ds(i * L, L)]
            sel = jnp.logical_and((iota + i * L) < m, v > lo)
            return (
                ka + jnp.where(sel, 1.0, 0.0),
                sa + jnp.where(sel, v, 0.0),
            )

        ka, sa = lax.fori_loop(0, nchunks(m), step, (vzero, vzero))
        return jnp.sum(ka), jnp.sum(sa)

    def g_dense(buf, t):
        def step(i, acc):
            v = buf[pl.ds(i * L, L)]
            return acc + jnp.maximum(v - t, 0.0)

        acc = lax.fori_loop(0, NCHUNK, step, vzero, unroll=4)
        return jnp.sum(acc)

    def ks_dense(buf, lo):
        def step(i, acc):
            ka, sa = acc
            v = buf[pl.ds(i * L, L)]
            sel = v > lo
            return (
                ka + jnp.where(sel, 1.0, 0.0),
                sa + jnp.where(sel, v, 0.0),
            )

        ka, sa = lax.fori_loop(0, NCHUNK, step, (vzero, vzero))
        return jnp.sum(ka), jnp.sum(sa)

    def solve(g_fn, ks_fn, lo0, hi0):
        def bis(_, lh):
            lo, hi = lh
            mid = 0.5 * (lo + hi)
            ge = g_fn(mid) >= 1.0
            return (jnp.where(ge, mid, lo), jnp.where(ge, hi, mid))

        lo, _ = lax.fori_loop(
            0,
            BISECT_ITERS,
            bis,
            (jnp.broadcast_to(lo0, (L,)), jnp.broadcast_to(hi0, (L,))),
        )
        kk, ss = ks_fn(lo)
        return (jnp.broadcast_to(ss, (L,)) - 1.0) / jnp.broadcast_to(kk, (L,))

    def dense_relu(buf, t):
        def step(i):
            ob[pl.ds(i * L, L)] = jnp.maximum(buf[pl.ds(i * L, L)] - t, 0.0)

        plsc.parallel_loop(0, NCHUNK, 1, unroll=8)(step)

    wid = lax.axis_index("s") * NC + lax.axis_index("c")
    row0 = wid * ROWS_PER
    bufs = (in0, in1)
    sems = (s0, s1)

    in_handles = {0: pltpu.async_copy(x_hbm.at[row0], in0, s0)}
    out_handle = None

    for j in range(ROWS_PER):
        buf = bufs[j % 2]
        in_handles[j].wait()
        if j + 1 < ROWS_PER:
            in_handles[j + 1] = pltpu.async_copy(
                x_hbm.at[row0 + j + 1], bufs[(j + 1) % 2], sems[(j + 1) % 2]
            )

        iota = lax.iota(jnp.int32, L)
        smax = sampled_max(buf)
        cut_v = jnp.broadcast_to(smax - 1.0, (L,))
        m, xmax = compress(buf, cut_v)
        cutoff = xmax - 1.0
        fast = m <= FAST_MAX

        def fast_fn(opnd, iota=iota):
            mm, cut, xm = opnd
            return solve(
                lambda tt: g_cand(iota, mm, tt),
                lambda lo: ks_cand(iota, mm, lo),
                cut,
                xm,
            )

        def slow_fn(opnd, buf=buf):
            mm, cut, xm = opnd
            return solve(
                lambda tt: g_dense(buf, tt),
                lambda lo: ks_dense(buf, lo),
                cut,
                xm,
            )

        t = lax.cond(fast, fast_fn, slow_fn, (m, cutoff, xmax))

        if out_handle is not None:
            out_handle.wait()
        dense_relu(buf, t)
        out_handle = pltpu.async_copy(ob, o_hbm.at[row0 + j], so)

    out_handle.wait()


def kernel(x):
    return _sparsemax_sc(x)


# speedup vs baseline: 37.3363x; 1.0281x over previous
import functools

import jax
import jax.numpy as jnp
from jax import lax
from jax.experimental import pallas as pl
from jax.experimental.pallas import tpu as pltpu
from jax.experimental.pallas import tpu_sc as plsc

L = 16
NROWS = 128
DIM = 32768
NCHUNK = DIM // L
NC = 2
NS = 16
NW = NC * NS
ROWS_PER = NROWS // NW
CAP = 16376
FAST_MAX = CAP - L
BISECT_ITERS = 24


_mesh = plsc.VectorSubcoreMesh(
    core_axis_name="c", subcore_axis_name="s", num_cores=NC, num_subcores=NS
)


@functools.partial(
    pl.kernel,
    out_type=jax.ShapeDtypeStruct((NROWS, DIM), jnp.float32),
    mesh=_mesh,
    scratch_types=[
        pltpu.VMEM((DIM,), jnp.float32),
        pltpu.VMEM((DIM,), jnp.float32),
        pltpu.VMEM((DIM,), jnp.float32),
        pltpu.VMEM((CAP,), jnp.float32),
        pltpu.SemaphoreType.DMA,
        pltpu.SemaphoreType.DMA,
        pltpu.SemaphoreType.DMA,
    ],
    compiler_params=pltpu.CompilerParams(needs_layout_passes=False),
)
def _sparsemax_sc(x_hbm, o_hbm, in0, in1, ob, cand_v, s0, s1, so):
    vzero = jnp.zeros((L,), jnp.float32)
    vone_i = jnp.ones((L,), jnp.int32)

    def sampled_max(buf):
        def gstep(g, acc):
            a = jnp.maximum(buf[pl.ds(g * 8 * 2 * L, L)],
                            buf[pl.ds((g * 8 * 2 + 8) * L, L)])
            return jnp.maximum(acc, a)

        acc = plsc.parallel_loop(
            0, NCHUNK // 16, 1, unroll=4,
            carry=jnp.full((L,), -jnp.inf, jnp.float32),
        )(gstep)
        return jnp.max(acc)

    def compress(buf, cutoff):
        def step(i, carry):
            off_m1, mx = carry
            v = buf[pl.ds(i * L, L)]
            mask = v > cutoff
            prefix = plsc.cumsum(vone_i, mask=mask)
            dest = off_m1 + prefix
            dest = plsc.bitcast(
                jnp.minimum(
                    plsc.bitcast(dest, jnp.uint32), jnp.uint32(CAP - 1)
                ),
                jnp.int32,
            )
            plsc.store_scatter(cand_v, [dest], v, mask=mask)
            cnt = plsc.all_reduce_population_count(mask)
            return off_m1 + cnt, jnp.maximum(mx, v)

        off_m1, mx = plsc.parallel_loop(
            0, NCHUNK, 1, unroll=8,
            carry=(
                jnp.full((L,), -1, jnp.int32),
                jnp.full((L,), -jnp.inf, jnp.float32),
            ),
        )(step)
        return off_m1[0] + 1, jnp.max(mx)

    def nchunks(m):
        return (m + (L - 1)) >> 4

    def g_cand(iota, m, t):
        def step(i, acc):
            v = cand_v[pl.ds(i * L, L)]
            valid = (iota + i * L) < m
            return acc + jnp.where(valid, jnp.maximum(v - t, 0.0), 0.0)

        acc = lax.fori_loop(0, nchunks(m), step, vzero)
        return jnp.sum(acc)

    def ks_cand(iota, m, lo):
        def step(i, acc):
            ka, sa = acc
            v = cand_v[pl.ds(i * L, L)]
            sel = jnp.logical_and((iota + i * L) < m, v > lo)
            return (
                ka + jnp.where(sel, 1.0, 0.0),
                sa + jnp.where(sel, v, 0.0),
            )

        ka, sa = lax.fori_loop(0, nchunks(m), step, (vzero, vzero))
        return jnp.sum(ka), jnp.sum(sa)

    def g_dense(buf, t):
        def step(i, acc):
            v = buf[pl.ds(i * L, L)]
            return acc + jnp.maximum(v - t, 0.0)

        acc = lax.fori_loop(0, NCHUNK, step, vzero, unroll=4)
        return jnp.sum(acc)

    def ks_dense(buf, lo):
        def step(i, acc):
            ka, sa = acc
            v = buf[pl.ds(i * L, L)]
            sel = v > lo
            return (
                ka + jnp.where(sel, 1.0, 0.0),
                sa + jnp.where(sel, v, 0.0),
            )

        ka, sa = lax.fori_loop(0, NCHUNK, step, (vzero, vzero))
        return jnp.sum(ka), jnp.sum(sa)

    def solve(g_fn, ks_fn, lo0, hi0):
        def bis(_, lh):
            lo, hi = lh
            mid = 0.5 * (lo + hi)
            ge = g_fn(mid) >= 1.0
            return (jnp.where(ge, mid, lo), jnp.where(ge, hi, mid))

        lo, _ = lax.fori_loop(
            0,
            BISECT_ITERS,
            bis,
            (jnp.broadcast_to(lo0, (L,)), jnp.broadcast_to(hi0, (L,))),
        )
        kk, ss = ks_fn(lo)
        return (jnp.broadcast_to(ss, (L,)) - 1.0) / jnp.broadcast_to(kk, (L,))

    def dense_relu(buf, t):
        def step(i):
            ob[pl.ds(i * L, L)] = jnp.maximum(buf[pl.ds(i * L, L)] - t, 0.0)

        plsc.parallel_loop(0, NCHUNK, 1, unroll=8)(step)

    wid = lax.axis_index("s") * NC + lax.axis_index("c")
    row0 = wid * ROWS_PER
    bufs = (in0, in1)
    sems = (s0, s1)

    in_handles = {0: pltpu.async_copy(x_hbm.at[row0], in0, s0)}
    out_handle = None

    for j in range(ROWS_PER):
        buf = bufs[j % 2]
        in_handles[j].wait()
        if j + 1 < ROWS_PER:
            in_handles[j + 1] = pltpu.async_copy(
                x_hbm.at[row0 + j + 1], bufs[(j + 1) % 2], sems[(j + 1) % 2]
            )

        iota = lax.iota(jnp.int32, L)
        smax = sampled_max(buf)
        cut_v = jnp.broadcast_to(smax - 1.0, (L,))
        m, xmax = compress(buf, cut_v)
        cutoff = xmax - 1.0
        fast = m <= FAST_MAX

        def fast_fn(opnd, iota=iota):
            mm, cut, xm = opnd
            return solve(
                lambda tt: g_cand(iota, mm, tt),
                lambda lo: ks_cand(iota, mm, lo),
                cut,
                xm,
            )

        def slow_fn(opnd, buf=buf):
            mm, cut, xm = opnd
            return solve(
                lambda tt: g_dense(buf, tt),
                lambda lo: ks_dense(buf, lo),
                cut,
                xm,
            )

        t = lax.cond(fast, fast_fn, slow_fn, (m, cutoff, xmax))

        if out_handle is not None:
            out_handle.wait()
        dense_relu(buf, t)
        out_handle = pltpu.async_copy(ob, o_hbm.at[row0 + j], so)

    out_handle.wait()


def kernel(x):
    return _sparsemax_sc(x)
